# baseline (speedup 1.0000x reference)
import jax
import jax.numpy as jnp
from jax import lax
from jax.experimental import pallas as pl

N = 48
C = 32
D = 4
H = 8
DK = 8
M = 16
L = 4
F = 16

_DEG_MS = [[0], [1, 2, 3], [4, 5, 6, 7, 8], [9, 10, 11, 12, 13, 14, 15]]
_RSQRT_DK = 0.3535533905932738


def _body(h_ref, x_ref, W_in_ref, Wq_ref, Wk_ref, Wv_ref, rW1_ref, rb1_ref,
          rW2_ref, nw_ref, nb_ref, W_out_ref, b_out_ref, out_ref):
    f32 = jnp.float32
    hg = h_ref[0]
    xg = x_ref[0]
    rel = xg[None, :, :] - xg[:, None, :]
    d2 = jnp.sum(rel * rel, axis=2, keepdims=True)
    dist = jnp.sqrt(d2 + 1e-8)
    rhat = rel / dist
    sel = lax.broadcasted_iota(jnp.int32, (1, 1, 3), 2)
    xc = jnp.sum(jnp.where(sel == 0, rhat, 0.0), axis=2, keepdims=True)
    yc = jnp.sum(jnp.where(sel == 1, rhat, 0.0), axis=2, keepdims=True)
    zc = jnp.sum(jnp.where(sel == 2, rhat, 0.0), axis=2, keepdims=True)
    one = jnp.ones((N, N, 1), f32)
    Ys = [one,
          xc, yc, zc,
          xc * yc, yc * zc, 3.0 * zc * zc - 1.0, xc * zc, xc * xc - yc * yc,
          yc * (3.0 * xc * xc - yc * yc), xc * yc * zc,
          yc * (5.0 * zc * zc - 1.0), zc * (5.0 * zc * zc - 3.0),
          xc * (5.0 * zc * zc - 1.0), zc * (xc * xc - yc * yc),
          xc * (xc * xc - 3.0 * yc * yc)]
    centers = lax.broadcasted_iota(jnp.int32, (1, 1, 16), 2).astype(f32) \
        * (6.0 / 15.0)
    rbf3 = jnp.exp(-((dist - centers) ** 2) / 0.5)
    rbf2 = rbf3.reshape(N * N, 16)
    diag = (lax.broadcasted_iota(jnp.int32, (N, N, 1), 0)
            == lax.broadcasted_iota(jnp.int32, (N, N, 1), 1))
    G = (lax.broadcasted_iota(jnp.int32, (H * DK, H), 0) // DK
         == lax.broadcasted_iota(jnp.int32, (H * DK, H), 1)).astype(f32)
    R = (lax.broadcasted_iota(jnp.int32, (H, C), 0)
         == lax.broadcasted_iota(jnp.int32, (H, C), 1) // (C // H)).astype(f32)
    S = (lax.broadcasted_iota(jnp.int32, (N, N * N), 1) // N
         == lax.broadcasted_iota(jnp.int32, (N, N * N), 0)).astype(f32)

    Ydeg = [jnp.concatenate([Ys[m] for m in _DEG_MS[l]], axis=2)
            for l in range(D)]

    f0 = jnp.dot(hg, W_in_ref[...], preferred_element_type=f32)
    feats = [f0] + [jnp.zeros((N, C), f32) for _ in range(M - 1)]

    for li in range(L):
        invs = []
        for l in range(D):
            s = feats[_DEG_MS[l][0]] * feats[_DEG_MS[l][0]]
            for m in _DEG_MS[l][1:]:
                s = s + feats[m] * feats[m]
            invs.append(jnp.sqrt(s + 1e-8))
        inv = jnp.concatenate(invs, axis=1)
        Wk_l = Wk_ref[li]
        q = jnp.dot(inv, Wq_ref[li],
                    preferred_element_type=f32) * _RSQRT_DK
        kn = jnp.dot(inv, Wk_l[0:D * C, :], preferred_element_type=f32)
        ke = jnp.dot(rbf2, Wk_l[D * C:, :], preferred_element_type=f32)
        kfull = kn[None, :, :] + ke.reshape(N, N, H * DK)
        qG = q[:, :, None] * G[None, :, :]
        logits = lax.dot_general(kfull, qG, (((2,), (1,)), ((0,), (0,))),
                                 preferred_element_type=f32)
        logits = jnp.where(diag, -1e30, logits)
        mx = jnp.max(logits, axis=1, keepdims=True)
        ex = jnp.exp(logits - mx)
        den = jnp.sum(ex, axis=1, keepdims=True)
        attn = ex / (den + 1e-9)
        attn_c = jnp.dot(attn.reshape(N * N, H), R,
                         preferred_element_type=f32).reshape(N, N, C)
        hid = jnp.maximum(
            jnp.dot(rbf2, rW1_ref[li], preferred_element_type=f32)
            + rb1_ref[li:li + 1, :], 0.0)
        rad3 = jnp.dot(hid, rW2_ref[li],
                       preferred_element_type=f32).reshape(N, N, D * C)
        f0s = feats[0][None, :, :]
        PFbase = attn_c * f0s
        new_feats = [None] * M
        for l in range(D):
            PF = PFbase * rad3[:, :, l * C:(l + 1) * C]
            Wv_l = Wv_ref[li, l]
            agg2 = lax.dot_general(Ydeg[l], PF,
                                   (((1,), (1,)), ((0,), (0,))),
                                   preferred_element_type=f32)
            fls = []
            for k, m in enumerate(_DEG_MS[l]):
                vn = jnp.dot(feats[m], Wv_l, preferred_element_type=f32)
                pay = attn_c * vn[None, :, :]
                agg = jnp.dot(S, pay.reshape(N * N, C),
                              preferred_element_type=f32) \
                    + agg2[:, k, :]
                fls.append(feats[m] + agg)
            s = fls[0] * fls[0]
            for f in fls[1:]:
                s = s + f * f
            nrm = jnp.sqrt(s + 1e-8)
            phi = jnp.maximum(
                jnp.dot(nrm, nw_ref[li, l], preferred_element_type=f32)
                + nb_ref[li, l:l + 1, :], 0.0)
            scale = phi / (nrm + 1e-8)
            for idx, m in enumerate(_DEG_MS[l]):
                new_feats[m] = fls[idx] * scale
        feats = new_feats

    invs = []
    for l in range(D):
        s = feats[_DEG_MS[l][0]] * feats[_DEG_MS[l][0]]
        for m in _DEG_MS[l][1:]:
            s = s + feats[m] * feats[m]
        invs.append(jnp.sqrt(s + 1e-8))
    inv = jnp.concatenate(invs, axis=1)
    node_out = jnp.dot(inv, W_out_ref[...],
                       preferred_element_type=f32) + b_out_ref[0:1, :]
    out_ref[0] = jnp.mean(node_out, axis=0, keepdims=True)


def kernel(h, x, W_in, Wq, Wk, Wv, rad_W1, rad_b1, rad_W2, nw, nb, W_out,
           b_out):
    B = h.shape[0]
    specs = [
        pl.BlockSpec((1, N, F), lambda b: (b, 0, 0)),
        pl.BlockSpec((1, N, 3), lambda b: (b, 0, 0)),
        pl.BlockSpec(W_in.shape, lambda b: (0, 0)),
        pl.BlockSpec(Wq.shape, lambda b: (0, 0, 0)),
        pl.BlockSpec(Wk.shape, lambda b: (0, 0, 0)),
        pl.BlockSpec(Wv.shape, lambda b: (0, 0, 0, 0)),
        pl.BlockSpec(rad_W1.shape, lambda b: (0, 0, 0)),
        pl.BlockSpec(rad_b1.shape, lambda b: (0, 0)),
        pl.BlockSpec(rad_W2.shape, lambda b: (0, 0, 0)),
        pl.BlockSpec(nw.shape, lambda b: (0, 0, 0, 0)),
        pl.BlockSpec(nb.shape, lambda b: (0, 0, 0)),
        pl.BlockSpec(W_out.shape, lambda b: (0, 0)),
        pl.BlockSpec((1, D * C), lambda b: (0, 0)),
    ]
    out = pl.pallas_call(
        _body,
        grid=(B,),
        in_specs=specs,
        out_specs=pl.BlockSpec((1, 1, D * C), lambda b: (b, 0, 0)),
        out_shape=jax.ShapeDtypeStruct((B, 1, D * C), jnp.float32),
    )(h, x, W_in, Wq, Wk, Wv, rad_W1, rad_b1, rad_W2, nw, nb, W_out,
      b_out.reshape(1, D * C))
    return out.reshape(B, D * C)


# speedup vs baseline: 1.0652x; 1.0652x over previous
import jax
import jax.numpy as jnp
from jax import lax
from jax.experimental import pallas as pl

N = 48
C = 32
D = 4
H = 8
DK = 8
M = 16
L = 4
F = 16

_DEG_MS = [[0], [1, 2, 3], [4, 5, 6, 7, 8], [9, 10, 11, 12, 13, 14, 15]]
_RSQRT_DK = 0.3535533905932738


def _body(h_ref, x_ref, W_in_ref, Wq_ref, Wk_ref, Wv_ref, rW1_ref, rb1_ref,
          rW2_ref, nw_ref, nb_ref, W_out_ref, b_out_ref, out_ref):
    f32 = jnp.float32
    hg = h_ref[0]
    xg = x_ref[0]
    rel = xg[None, :, :] - xg[:, None, :]
    d2 = jnp.sum(rel * rel, axis=2, keepdims=True)
    dist = jnp.sqrt(d2 + 1e-8)
    rhat = rel / dist
    sel = lax.broadcasted_iota(jnp.int32, (1, 1, 3), 2)
    xc = jnp.sum(jnp.where(sel == 0, rhat, 0.0), axis=2, keepdims=True)
    yc = jnp.sum(jnp.where(sel == 1, rhat, 0.0), axis=2, keepdims=True)
    zc = jnp.sum(jnp.where(sel == 2, rhat, 0.0), axis=2, keepdims=True)
    one = jnp.ones((N, N, 1), f32)
    Ys = [one,
          xc, yc, zc,
          xc * yc, yc * zc, 3.0 * zc * zc - 1.0, xc * zc, xc * xc - yc * yc,
          yc * (3.0 * xc * xc - yc * yc), xc * yc * zc,
          yc * (5.0 * zc * zc - 1.0), zc * (5.0 * zc * zc - 3.0),
          xc * (5.0 * zc * zc - 1.0), zc * (xc * xc - yc * yc),
          xc * (xc * xc - 3.0 * yc * yc)]
    centers = lax.broadcasted_iota(jnp.int32, (1, 1, 16), 2).astype(f32) \
        * (6.0 / 15.0)
    rbf3 = jnp.exp(-((dist - centers) ** 2) / 0.5)
    rbf2 = rbf3.reshape(N * N, 16)
    diag = (lax.broadcasted_iota(jnp.int32, (N, N, 1), 0)
            == lax.broadcasted_iota(jnp.int32, (N, N, 1), 1))
    G = (lax.broadcasted_iota(jnp.int32, (H * DK, H), 0) // DK
         == lax.broadcasted_iota(jnp.int32, (H * DK, H), 1)).astype(f32)
    R = (lax.broadcasted_iota(jnp.int32, (H, C), 0)
         == lax.broadcasted_iota(jnp.int32, (H, C), 1) // (C // H)).astype(f32)
    S = (lax.broadcasted_iota(jnp.int32, (N, N * N), 1) // N
         == lax.broadcasted_iota(jnp.int32, (N, N * N), 0)).astype(f32)

    Ydeg = [jnp.concatenate([Ys[m] for m in _DEG_MS[l]], axis=2)
            for l in range(D)]

    f0 = jnp.dot(hg, W_in_ref[...], preferred_element_type=f32)
    feats = [f0] + [jnp.zeros((N, C), f32) for _ in range(M - 1)]

    for li in range(L):
        invs = []
        for l in range(D):
            s = feats[_DEG_MS[l][0]] * feats[_DEG_MS[l][0]]
            for m in _DEG_MS[l][1:]:
                s = s + feats[m] * feats[m]
            invs.append(jnp.sqrt(s + 1e-8))
        inv = jnp.concatenate(invs, axis=1)
        Wk_l = Wk_ref[li]
        q = jnp.dot(inv, Wq_ref[li], preferred_element_type=f32)
        kn = jnp.dot(inv, Wk_l[0:D * C, :], preferred_element_type=f32)
        ke = jnp.dot(rbf2, Wk_l[D * C:, :], preferred_element_type=f32)
        qk = q[:, None, :] * (kn[None, :, :] + ke.reshape(N, N, H * DK))
        logits = jnp.dot(qk.reshape(N * N, H * DK), G,
                         preferred_element_type=f32).reshape(N, N, H)
        logits = jnp.where(diag, -1e30, logits * _RSQRT_DK)
        mx = jnp.max(logits, axis=1, keepdims=True)
        ex = jnp.exp(logits - mx)
        den = jnp.sum(ex, axis=1, keepdims=True)
        attn = ex * (1.0 / (den + 1e-9))
        attn_c = jnp.dot(attn.reshape(N * N, H), R,
                         preferred_element_type=f32).reshape(N, N, C)
        hid = jnp.maximum(
            jnp.dot(rbf2, rW1_ref[li], preferred_element_type=f32)
            + rb1_ref[li:li + 1, :], 0.0)
        rad3 = jnp.dot(hid, rW2_ref[li],
                       preferred_element_type=f32).reshape(N, N, D * C)
        f0s = feats[0][None, :, :]
        PFbase = attn_c * f0s
        new_feats = [None] * M
        for l in range(D):
            PF = PFbase * rad3[:, :, l * C:(l + 1) * C]
            Wv_l = Wv_ref[li, l]
            agg2 = lax.dot_general(Ydeg[l], PF,
                                   (((1,), (1,)), ((0,), (0,))),
                                   preferred_element_type=f32)
            fls = []
            for k, m in enumerate(_DEG_MS[l]):
                vn = jnp.dot(feats[m], Wv_l, preferred_element_type=f32)
                pay = attn_c * vn[None, :, :]
                agg = jnp.dot(S, pay.reshape(N * N, C),
                              preferred_element_type=f32) \
                    + agg2[:, k, :]
                fls.append(feats[m] + agg)
            s = fls[0] * fls[0]
            for f in fls[1:]:
                s = s + f * f
            nrm = jnp.sqrt(s + 1e-8)
            phi = jnp.maximum(
                jnp.dot(nrm, nw_ref[li, l], preferred_element_type=f32)
                + nb_ref[li, l:l + 1, :], 0.0)
            scale = phi / (nrm + 1e-8)
            for idx, m in enumerate(_DEG_MS[l]):
                new_feats[m] = fls[idx] * scale
        feats = new_feats

    invs = []
    for l in range(D):
        s = feats[_DEG_MS[l][0]] * feats[_DEG_MS[l][0]]
        for m in _DEG_MS[l][1:]:
            s = s + feats[m] * feats[m]
        invs.append(jnp.sqrt(s + 1e-8))
    inv = jnp.concatenate(invs, axis=1)
    node_out = jnp.dot(inv, W_out_ref[...],
                       preferred_element_type=f32) + b_out_ref[0:1, :]
    out_ref[0] = jnp.mean(node_out, axis=0, keepdims=True)


def kernel(h, x, W_in, Wq, Wk, Wv, rad_W1, rad_b1, rad_W2, nw, nb, W_out,
           b_out):
    B = h.shape[0]
    specs = [
        pl.BlockSpec((1, N, F), lambda b: (b, 0, 0)),
        pl.BlockSpec((1, N, 3), lambda b: (b, 0, 0)),
        pl.BlockSpec(W_in.shape, lambda b: (0, 0)),
        pl.BlockSpec(Wq.shape, lambda b: (0, 0, 0)),
        pl.BlockSpec(Wk.shape, lambda b: (0, 0, 0)),
        pl.BlockSpec(Wv.shape, lambda b: (0, 0, 0, 0)),
        pl.BlockSpec(rad_W1.shape, lambda b: (0, 0, 0)),
        pl.BlockSpec(rad_b1.shape, lambda b: (0, 0)),
        pl.BlockSpec(rad_W2.shape, lambda b: (0, 0, 0)),
        pl.BlockSpec(nw.shape, lambda b: (0, 0, 0, 0)),
        pl.BlockSpec(nb.shape, lambda b: (0, 0, 0)),
        pl.BlockSpec(W_out.shape, lambda b: (0, 0)),
        pl.BlockSpec((1, D * C), lambda b: (0, 0)),
    ]
    out = pl.pallas_call(
        _body,
        grid=(B,),
        in_specs=specs,
        out_specs=pl.BlockSpec((1, 1, D * C), lambda b: (b, 0, 0)),
        out_shape=jax.ShapeDtypeStruct((B, 1, D * C), jnp.float32),
    )(h, x, W_in, Wq, Wk, Wv, rad_W1, rad_b1, rad_W2, nw, nb, W_out,
      b_out.reshape(1, D * C))
    return out.reshape(B, D * C)
